# baseline (device time: 15026 ns/iter reference)
import jax
import jax.numpy as jnp
from jax import lax
from jax.experimental import pallas as pl
from jax.experimental.pallas import tpu as pltpu

N_DEV = 4
B, Sq, Hq, Dh = 2, 256, 4, 64
SKV_SHARD = 256
D = Hq * Dh
R = Sq + Hq

_RANGES = [
    [(0, R)],
    [(0, 32), (128, R - 128)],
    [(0, 32), (Sq, Hq)],
    [(0, 32), (Sq, Hq)],
]


def kernel(x, Wq, K_ext, V_ext, Wo):
    Kp = jnp.transpose(K_ext, (0, 2, 1, 3)).reshape(
        B * Hq, SKV_SHARD, Dh).astype(jnp.bfloat16)
    Vp = jnp.transpose(V_ext, (0, 2, 1, 3)).reshape(
        B * Hq, SKV_SHARD, Dh).astype(jnp.bfloat16)
    x = x.astype(jnp.bfloat16)
    Wq = Wq.astype(jnp.bfloat16)
    Wo = Wo.astype(jnp.bfloat16)

    def body(x_ref, wq_ref, k_ref, v_ref, wo_ref, out_ref,
             sbuf, rbuf, send_sems, recv_sems):
        my_pos = lax.axis_index("i")
        peers = [
            lax.rem(my_pos + 1, N_DEV),
            lax.rem(my_pos - 1 + N_DEV, N_DEV),
            lax.rem(my_pos + 2, N_DEV),
        ]
        origin_of = lambda c, t: [(c - 1) % N_DEV, (c + 1) % N_DEV,
                                  (c + 2) % N_DEV][t]

        rbuf[...] = jnp.zeros((N_DEV - 1, B, R, D), jnp.bfloat16)

        barrier_sem = pltpu.get_barrier_semaphore()
        for p in peers:
            pl.semaphore_signal(
                barrier_sem, inc=1,
                device_id=(p,), device_id_type=pl.DeviceIdType.MESH,
            )

        qi = lax.broadcasted_iota(jnp.int32, (Sq, SKV_SHARD), 0)
        kg = lax.broadcasted_iota(jnp.int32, (Sq, SKV_SHARD), 1) \
            + my_pos * SKV_SHARD
        mask = (jnp.abs(qi - kg) <= 128) | (kg < 32) | (qi < 32)

        def descriptor(b, t, p, j, r0, n):
            return pltpu.make_async_remote_copy(
                src_ref=sbuf.at[b, pl.ds(r0, n)],
                dst_ref=rbuf.at[t, b, pl.ds(r0, n)],
                send_sem=send_sems.at[b, t, j],
                recv_sem=recv_sems.at[b, t, j],
                device_id=(p,), device_id_type=pl.DeviceIdType.MESH,
            )

        for b in range(B):
            qb = jnp.dot(x_ref[b], wq_ref[...],
                         preferred_element_type=jnp.float32)
            for h in range(Hq):
                q = qb[:, h * Dh:(h + 1) * Dh].astype(jnp.bfloat16)
                sc = jax.lax.dot_general(
                    q, k_ref[b * Hq + h],
                    (((1,), (1,)), ((), ())),
                    preferred_element_type=jnp.float32,
                ) * 0.125
                e = jnp.where(mask, jnp.exp(sc), 0.0)
                sbuf[b, Sq + h] = jnp.sum(e, axis=1).astype(jnp.bfloat16)
                sbuf[b, 0:Sq, h * Dh:(h + 1) * Dh] = jnp.dot(
                    e.astype(jnp.bfloat16), v_ref[b * Hq + h],
                    preferred_element_type=jnp.float32,
                ).astype(jnp.bfloat16)
            if b == 0:
                pl.semaphore_wait(barrier_sem, N_DEV - 1)
            for c in range(N_DEV):
                @pl.when(my_pos == c)
                def _(b=b, c=c):
                    for t, p in enumerate(peers):
                        for j, (r0, n) in enumerate(_RANGES[c]):
                            descriptor(b, t, p, j, r0, n).start()

        for b in range(B):
            for c in range(N_DEV):
                @pl.when(my_pos == c)
                def _(b=b, c=c):
                    for t, p in enumerate(peers):
                        for j, (r0, n) in enumerate(_RANGES[origin_of(c, t)]):
                            descriptor(b, t, p, j, r0, n).wait_recv()
            ctx = sbuf[b, 0:Sq, :].astype(jnp.float32)
            den = sbuf[b, Sq:R, :].astype(jnp.float32)
            for t in range(N_DEV - 1):
                ctx = ctx + rbuf[t, b, 0:Sq, :].astype(jnp.float32)
                den = den + rbuf[t, b, Sq:R, :].astype(jnp.float32)
            div = jnp.broadcast_to(
                jnp.transpose(den)[:, :, None], (Sq, Hq, Dh)
            ).reshape(Sq, D)
            out_ref[b] = jnp.dot((ctx / div).astype(jnp.bfloat16),
                                 wo_ref[...],
                                 preferred_element_type=jnp.float32)

        for c in range(N_DEV):
            @pl.when(my_pos == c)
            def _(c=c):
                for b in range(B):
                    for t, p in enumerate(peers):
                        for j, (r0, n) in enumerate(_RANGES[c]):
                            descriptor(b, t, p, j, r0, n).wait_send()

    return pl.pallas_call(
        body,
        out_shape=jax.ShapeDtypeStruct((B, Sq, 512), jnp.float32),
        in_specs=[pl.BlockSpec(memory_space=pltpu.VMEM)] * 5,
        out_specs=pl.BlockSpec(memory_space=pltpu.VMEM),
        scratch_shapes=[
            pltpu.VMEM((B, R, D), jnp.bfloat16),
            pltpu.VMEM((N_DEV - 1, B, R, D), jnp.bfloat16),
            pltpu.SemaphoreType.DMA((B, N_DEV - 1, 2)),
            pltpu.SemaphoreType.DMA((B, N_DEV - 1, 2)),
        ],
        compiler_params=pltpu.CompilerParams(collective_id=0),
    )(x, Wq, Kp, Vp, Wo)
